# baseline (speedup 1.0000x reference)
import functools

import jax
import jax.numpy as jnp
from jax import lax
from jax.experimental import pallas as pl
from jax.experimental.pallas import tpu as pltpu
from jax.experimental.pallas import tpu_sc as plsc

VOCAB = 100000
D = 128
B = 1024
L = 200
LANES = 16
NC = 2
NS = 16
NW = NC * NS
SEQ_PER_W = B // NW
ROWS_PER_W = SEQ_PER_W * L
SCALE = float(D) ** 0.5
CH = 40
NCHUNK = ROWS_PER_W // CH
NBUF = 6
PREF = 3


def _body(seq_hbm, table_hbm, pos_hbm, out_hbm,
          idx_all, pos_v, rows_v, sem_g, sem_w):
    wid = lax.axis_index("s") * NC + lax.axis_index("c")
    wbase = wid * ROWS_PER_W
    pltpu.sync_copy(pos_hbm.at[0, pl.ds(0, L)], pos_v)
    pltpu.sync_copy(seq_hbm.at[pl.ds(wbase, ROWS_PER_W)], idx_all)

    def chunk_refs(i, k):
        idx = idx_all.at[pl.ds(pl.multiple_of(i * CH, 8), CH)]
        dst = rows_v.at[k]
        out = out_hbm.at[pl.ds(pl.multiple_of(wbase + i * CH, 8), CH)]
        return idx, dst, out

    def gather_start(i, k):
        idx, dst, _ = chunk_refs(i, k)
        pltpu.async_copy(table_hbm.at[idx], dst, sem_g.at[k])

    def gather_wait(i, k):
        idx, dst, _ = chunk_refs(i, k)
        pltpu.make_async_copy(table_hbm.at[idx], dst, sem_g.at[k]).wait()

    def write_start(i, k):
        _, src, out = chunk_refs(i, k)
        pltpu.async_copy(src, out, sem_w.at[k])

    def write_wait(i, k):
        _, src, out = chunk_refs(i, k)
        pltpu.make_async_copy(src, out, sem_w.at[k]).wait()

    def compute(i, k):
        po = (i % (L // CH)) * CH

        def one_row(r, carry):
            for c in range(D // LANES):
                sl = pl.ds(c * LANES, LANES)
                rows_v[k, r, sl] = rows_v[k, r, sl] * SCALE + pos_v[po + r, sl]
            return carry

        lax.fori_loop(0, CH, one_row, 0, unroll=False)

    for t in range(PREF):
        gather_start(t, t)

    def step(i, carry):
        k = i % NBUF
        kn = (i + PREF) % NBUF

        @pl.when(i >= NBUF - PREF)
        def _():
            write_wait(i - (NBUF - PREF), kn)

        @pl.when(i + PREF < NCHUNK)
        def _():
            gather_start(i + PREF, kn)

        gather_wait(i, k)
        compute(i, k)
        write_start(i, k)
        return carry

    lax.fori_loop(0, NCHUNK, step, 0, unroll=False)
    for i in range(NCHUNK - (NBUF - PREF), NCHUNK):
        write_wait(i, i % NBUF)


@jax.jit
def _pre_process(sequence, emb_table, pos_encoding):
    f = functools.partial(
        pl.kernel,
        out_type=jax.ShapeDtypeStruct((B * L, D), jnp.float32),
        mesh=plsc.VectorSubcoreMesh(core_axis_name="c", subcore_axis_name="s"),
        scratch_types=[
            pltpu.VMEM((ROWS_PER_W,), jnp.int32),
            pltpu.VMEM((L, D), jnp.float32),
            pltpu.VMEM((NBUF, CH, D), jnp.float32),
            pltpu.SemaphoreType.DMA((NBUF,)),
            pltpu.SemaphoreType.DMA((NBUF,)),
        ],
    )(_body)
    return f(sequence, emb_table, pos_encoding)


def kernel(sequence, emb_table, pos_encoding, training=False, mask=None):
    seq = sequence.astype(jnp.int32).reshape(B * L)
    out = _pre_process(seq, emb_table, pos_encoding)
    return out.reshape(B, L, D)


# speedup vs baseline: 3.4663x; 3.4663x over previous
import functools

import jax
import jax.numpy as jnp
from jax import lax
from jax.experimental import pallas as pl
from jax.experimental.pallas import tpu as pltpu
from jax.experimental.pallas import tpu_sc as plsc

VOCAB = 100000
D = 128
B = 1024
L = 200
LANES = 16
NC = 2
NS = 16
NW = NC * NS
SEQ_PER_W = B // NW
SCALE = float(D) ** 0.5
LA, LB = 104, 96
NBUF = 6
NCHUNK = 2 * SEQ_PER_W
PREF = 4


def _body(seq_hbm, table_hbm, pos_hbm, out_hbm, idx_all, pos_v, rows_v,
          sg0, sg1, sg2, sg3, sg4, sg5, sw0, sw1, sw2, sw3, sw4, sw5):
    sem_g = [sg0, sg1, sg2, sg3, sg4, sg5]
    sem_w = [sw0, sw1, sw2, sw3, sw4, sw5]
    wid = lax.axis_index("s") * NC + lax.axis_index("c")
    base = wid * SEQ_PER_W
    pltpu.sync_copy(pos_hbm.at[0, pl.ds(0, L)], pos_v)
    pltpu.sync_copy(seq_hbm.at[pl.ds(base * L, SEQ_PER_W * L)], idx_all)

    def chunk_refs(i, t, parity):
        ln = LA if parity == 0 else LB
        seq = i // 2
        idx = idx_all.at[pl.ds(seq * L + parity * LA, ln)]
        dst = rows_v.at[t, pl.ds(0, ln)]
        out = out_hbm.at[base + seq, pl.ds(parity * LA, ln)]
        return idx, dst, out

    def gather_start(i, t, parity):
        idx, dst, _ = chunk_refs(i, t, parity)
        pltpu.async_copy(table_hbm.at[idx], dst, sem_g[t])

    def gather_wait(i, t, parity):
        idx, dst, _ = chunk_refs(i, t, parity)
        pltpu.make_async_copy(table_hbm.at[idx], dst, sem_g[t]).wait()

    def write_start(i, t, parity):
        _, src, out = chunk_refs(i, t, parity)
        pltpu.async_copy(src, out, sem_w[t])

    def write_wait(i, t, parity):
        _, src, out = chunk_refs(i, t, parity)
        pltpu.make_async_copy(src, out, sem_w[t]).wait()

    def compute(t, parity):
        ln = LA if parity == 0 else LB

        def one_row(r, carry):
            pr = r + parity * LA
            for c in range(D // LANES):
                sl = pl.ds(c * LANES, LANES)
                rows_v[t, r, sl] = rows_v[t, r, sl] * SCALE + pos_v[pr, sl]
            return carry

        lax.fori_loop(0, ln, one_row, 0, unroll=False)

    def step(i, t, parity, drain_write, prefetch):
        kn = (t + PREF) % NBUF
        pn = parity
        if drain_write:
            write_wait(i - (NBUF - PREF), kn, pn)
        if prefetch:
            gather_start(i + PREF, kn, pn)
        gather_wait(i, t, parity)
        compute(t, parity)
        write_start(i, t, parity)

    for t in range(PREF):
        gather_start(t, t, t % 2)
    for i in range(NBUF):
        step(i, i, i % 2, drain_write=(i >= NBUF - PREF), prefetch=True)

    def group(g, carry):
        for t in range(NBUF):
            step(g * NBUF + t, t, t % 2, drain_write=True, prefetch=True)
        return carry

    lax.fori_loop(1, (NCHUNK - PREF) // NBUF, group, 0, unroll=False)
    for i in range(NCHUNK - PREF, NCHUNK):
        step(i, i % NBUF, i % 2, drain_write=True, prefetch=False)
    for i in range(NCHUNK - (NBUF - PREF), NCHUNK):
        write_wait(i, i % NBUF, i % 2)


@jax.jit
def _pre_process(sequence, emb_table, pos_slice):
    f = functools.partial(
        pl.kernel,
        out_type=jax.ShapeDtypeStruct((B, L, D), jnp.float32),
        mesh=plsc.VectorSubcoreMesh(core_axis_name="c", subcore_axis_name="s"),
        scratch_types=[
            pltpu.VMEM((SEQ_PER_W * L,), jnp.int32),
            pltpu.VMEM((L, D), jnp.float32),
            pltpu.VMEM((NBUF, LA, D), jnp.float32),
        ] + [pltpu.SemaphoreType.DMA] * (2 * NBUF),
    )(_body)
    return f(sequence, emb_table, pos_slice)


def kernel(sequence, emb_table, pos_encoding, training=False, mask=None):
    seq = sequence.astype(jnp.int32).reshape(B * L)
    return _pre_process(seq, emb_table, pos_encoding)


# speedup vs baseline: 3.5212x; 1.0158x over previous
import functools

import jax
import jax.numpy as jnp
from jax import lax
from jax.experimental import pallas as pl
from jax.experimental.pallas import tpu as pltpu
from jax.experimental.pallas import tpu_sc as plsc

VOCAB = 100000
D = 128
B = 1024
L = 200
LANES = 16
NC = 2
NS = 16
NW = NC * NS
SEQ_PER_W = B // NW
SCALE = float(D) ** 0.5
LA, LB = 104, 96
NBUF = 6
NCHUNK = 2 * SEQ_PER_W
PREF = 4


def _body(seq_hbm, table_hbm, pos_hbm, out_hbm, idx_all, pos_v, rows_v,
          sg0, sg1, sg2, sg3, sg4, sg5, sw0, sw1, sw2, sw3, sw4, sw5, sp):
    sem_g = [sg0, sg1, sg2, sg3, sg4, sg5]
    sem_w = [sw0, sw1, sw2, sw3, sw4, sw5]
    wid = lax.axis_index("s") * NC + lax.axis_index("c")
    base = wid * SEQ_PER_W
    pltpu.sync_copy(seq_hbm.at[pl.ds(base * L, SEQ_PER_W * L)], idx_all)

    def chunk_refs(i, t, parity):
        ln = LA if parity == 0 else LB
        seq = i // 2
        idx = idx_all.at[pl.ds(seq * L + parity * LA, ln)]
        dst = rows_v.at[t, pl.ds(0, ln)]
        out = out_hbm.at[base + seq, pl.ds(parity * LA, ln)]
        return idx, dst, out

    def gather_start(i, t, parity):
        idx, dst, _ = chunk_refs(i, t, parity)
        pltpu.async_copy(table_hbm.at[idx], dst, sem_g[t])

    def gather_wait(i, t, parity):
        idx, dst, _ = chunk_refs(i, t, parity)
        pltpu.make_async_copy(table_hbm.at[idx], dst, sem_g[t]).wait()

    def write_start(i, t, parity):
        _, src, out = chunk_refs(i, t, parity)
        pltpu.async_copy(src, out, sem_w[t])

    def write_wait(i, t, parity):
        _, src, out = chunk_refs(i, t, parity)
        pltpu.make_async_copy(src, out, sem_w[t]).wait()

    def compute(t, parity):
        ln = LA if parity == 0 else LB

        def one_row(r, carry):
            pr = r + parity * LA
            for c in range(D // LANES):
                sl = pl.ds(c * LANES, LANES)
                rows_v[t, r, sl] = rows_v[t, r, sl] * SCALE + pos_v[pr, sl]
            return carry

        lax.fori_loop(0, ln, one_row, 0, unroll=False)

    def step(i, t, parity, drain_write, prefetch):
        kn = (t + PREF) % NBUF
        pn = parity
        if drain_write:
            write_wait(i - (NBUF - PREF), kn, pn)
        if prefetch:
            gather_start(i + PREF, kn, pn)
        gather_wait(i, t, parity)
        compute(t, parity)
        write_start(i, t, parity)

    for t in range(PREF):
        gather_start(t, t, t % 2)
    pltpu.async_copy(pos_hbm.at[0, pl.ds(0, L)], pos_v, sp).wait()
    for i in range(NBUF):
        step(i, i, i % 2, drain_write=(i >= NBUF - PREF), prefetch=True)

    def group(g, carry):
        for t in range(NBUF):
            step(g * NBUF + t, t, t % 2, drain_write=True, prefetch=True)
        return carry

    lax.fori_loop(1, (NCHUNK - PREF) // NBUF, group, 0, unroll=False)
    for i in range(NCHUNK - PREF, NCHUNK):
        step(i, i % NBUF, i % 2, drain_write=True, prefetch=False)
    for i in range(NCHUNK - (NBUF - PREF), NCHUNK):
        write_wait(i, i % NBUF, i % 2)


@jax.jit
def _pre_process(sequence, emb_table, pos_slice):
    f = functools.partial(
        pl.kernel,
        out_type=jax.ShapeDtypeStruct((B, L, D), jnp.float32),
        mesh=plsc.VectorSubcoreMesh(core_axis_name="c", subcore_axis_name="s"),
        scratch_types=[
            pltpu.VMEM((SEQ_PER_W * L,), jnp.int32),
            pltpu.VMEM((L, D), jnp.float32),
            pltpu.VMEM((NBUF, LA, D), jnp.float32),
        ] + [pltpu.SemaphoreType.DMA] * (2 * NBUF + 1),
    )(_body)
    return f(sequence, emb_table, pos_slice)


def kernel(sequence, emb_table, pos_encoding, training=False, mask=None):
    seq = sequence.astype(jnp.int32).reshape(B * L)
    return _pre_process(seq, emb_table, pos_encoding)
